# baseline (speedup 1.0000x reference)
import functools

import jax
import jax.numpy as jnp
from jax import lax
from jax.experimental import pallas as pl
from jax.experimental.pallas import tpu as pltpu
from jax.experimental.pallas import tpu_sc as plsc

NC = 2
NS = 16
L = 16
NW = NC * NS

B = 16384
D = 64
BPW = B // NW
CHUNK = 128
NCHUNK = BPW // CHUNK


def _lane_perm(v, idx):
    return lax.gather(
        v, idx[:, None],
        lax.GatherDimensionNumbers(offset_dims=(), collapsed_slice_dims=(0,),
                                   start_index_map=(0,)),
        (1,), mode=lax.GatherScatterMode.PROMISE_IN_BOUNDS)


def _rel_cmf_body(users_hbm, items_hbm, utab_hbm, itab_hbm,
                  uout_hbm, iout_hbm, r_hbm,
                  uidx_v, iidx_v, urows_v, irows_v, rhat_v, sem):
    wid = lax.axis_index("s") * NC + lax.axis_index("c")
    base = wid * BPW

    pltpu.sync_copy(users_hbm.at[pl.ds(base, BPW)], uidx_v)
    pltpu.sync_copy(items_hbm.at[pl.ds(base, BPW)], iidx_v)

    copies = []
    for j in range(NCHUNK):
        sl = pl.ds(j * CHUNK, CHUNK)
        copies.append(pltpu.async_copy(
            utab_hbm.at[uidx_v.at[sl]], urows_v.at[sl], sem))
        copies.append(pltpu.async_copy(
            itab_hbm.at[iidx_v.at[sl]], irows_v.at[sl], sem))
    for cp in copies:
        cp.wait()

    lane_iota = lax.iota(jnp.int32, L)
    perms = [lane_iota ^ sh for sh in (8, 4, 2, 1)]

    def group_body(g, carry):
        vec = jnp.zeros((L,), jnp.float32)
        for j in range(L):
            r = g * L + j
            acc = (urows_v[r, pl.ds(0, 16)] * irows_v[r, pl.ds(0, 16)]
                   + urows_v[r, pl.ds(16, 16)] * irows_v[r, pl.ds(16, 16)]
                   + urows_v[r, pl.ds(32, 16)] * irows_v[r, pl.ds(32, 16)]
                   + urows_v[r, pl.ds(48, 16)] * irows_v[r, pl.ds(48, 16)])
            for p in perms:
                acc = acc + _lane_perm(acc, p)
            vec = jnp.where(lane_iota == j, acc, vec)
        rhat_v[pl.ds(g * L, L)] = vec
        return carry

    lax.fori_loop(0, BPW // L, group_body, 0)

    pltpu.sync_copy(urows_v, uout_hbm.at[pl.ds(base, BPW)])
    pltpu.sync_copy(irows_v, iout_hbm.at[pl.ds(base, BPW)])
    pltpu.sync_copy(rhat_v, r_hbm.at[pl.ds(base, BPW)])


@jax.jit
def kernel(users, items, user_embeddings, item_embeddings):
    mesh = plsc.VectorSubcoreMesh(core_axis_name="c", subcore_axis_name="s")
    f = pl.kernel(
        _rel_cmf_body,
        mesh=mesh,
        out_type=(
            jax.ShapeDtypeStruct((B, D), jnp.float32),
            jax.ShapeDtypeStruct((B, D), jnp.float32),
            jax.ShapeDtypeStruct((B,), jnp.float32),
        ),
        scratch_types=[
            pltpu.VMEM((BPW,), jnp.int32),
            pltpu.VMEM((BPW,), jnp.int32),
            pltpu.VMEM((BPW, D), jnp.float32),
            pltpu.VMEM((BPW, D), jnp.float32),
            pltpu.VMEM((BPW,), jnp.float32),
            pltpu.SemaphoreType.DMA,
        ],
        compiler_params=pltpu.CompilerParams(use_tc_tiling_on_sc=False),
    )
    return f(users, items, user_embeddings, item_embeddings)
